# baseline (speedup 1.0000x reference)
import functools

import jax
import jax.numpy as jnp
from jax import lax
from jax.experimental import pallas as pl
from jax.experimental.pallas import tpu as pltpu
from jax.experimental.pallas import tpu_sc as plsc

N = 10000
E = 320000
D = 128
DH = D // 2
NPAD = 10240
NC = 2
NS = 16
ER = E // D
ERP = 2560
CH = 128

_MESH = plsc.VectorSubcoreMesh(
    core_axis_name="c", subcore_axis_name="s", num_cores=NC, num_subcores=NS)

_DEG_ROWS = ERP // NS
_NODES_PER_SUB = NPAD // (NC * NS)


def _sc_deg_body(row2d, dis128, idx_v, ones_v, zro_v, dstage_v, drows_v,
                 deg_sh, sem):
    c = lax.axis_index("c")
    s = lax.axis_index("s")

    z16 = jnp.zeros((16,), jnp.float32)
    for k in range(640 // 16):
        zro_v[pl.ds(k * 16, 16)] = z16
    o16 = jnp.ones((16,), jnp.float32)
    for k in range(CH // 16):
        ones_v[pl.ds(k * 16, 16)] = o16

    pltpu.sync_copy(zro_v, deg_sh.at[pl.ds(s * 640, 640)])

    pltpu.sync_copy(row2d.at[pl.ds(s * _DEG_ROWS, _DEG_ROWS), :], idx_v)

    plsc.subcore_barrier()

    def _scatter(j, carry):
        pltpu.sync_copy(ones_v, deg_sh.at[idx_v.at[j]], add=True)
        return carry

    lax.fori_loop(0, _DEG_ROWS, _scatter, 0, unroll=False)

    plsc.subcore_barrier()

    nbase = (c * NS + s) * _NODES_PER_SUB
    pltpu.sync_copy(deg_sh.at[pl.ds(nbase, _NODES_PER_SUB)], dstage_v)

    for v in range(_NODES_PER_SUB // 16):
        d = dstage_v[pl.ds(v * 16, 16)]
        i = lax.bitcast_convert_type(d, jnp.int32)
        i = jnp.int32(0x5F3759DF) - lax.shift_right_logical(i, 1)
        y = lax.bitcast_convert_type(i, jnp.float32)
        half = d * jnp.float32(0.5)
        for _ in range(3):
            y = y * (jnp.float32(1.5) - half * y * y)
        dstage_v[pl.ds(v * 16, 16)] = y

    def _splat(g, carry):
        v = dstage_v[pl.ds(g * 16, 16)]
        for r in range(16):
            v16 = lax.broadcast(v[r], (16,))
            for cc in range(D // 16):
                drows_v[r, pl.ds(cc * 16, 16)] = v16
        pltpu.sync_copy(drows_v, dis128.at[pl.ds(nbase + g * 16, 16), :])
        return carry

    lax.fori_loop(0, _NODES_PER_SUB // 16, _splat, 0, unroll=False)


_sc_deg = functools.partial(
    pl.kernel,
    out_type=jax.ShapeDtypeStruct((NPAD, D), jnp.float32),
    mesh=_MESH,
    scratch_types=[
        pltpu.VMEM((_DEG_ROWS, CH), jnp.int32),
        pltpu.VMEM((CH,), jnp.float32),
        pltpu.VMEM((640,), jnp.float32),
        pltpu.VMEM((_NODES_PER_SUB,), jnp.float32),
        pltpu.VMEM((16, D), jnp.float32),
        pltpu.VMEM_SHARED((NPAD,), jnp.float32),
        pltpu.SemaphoreType.DMA,
    ],
)(_sc_deg_body)


def _tc_main_body(x_ref, ws_ref, wn_ref, b_ref, dis_ref, self_ref, hps_ref):
    x = x_ref[...]
    self_ref[...] = (
        jnp.dot(x, ws_ref[...], preferred_element_type=jnp.float32)
        + b_ref[...])
    hp = dis_ref[...] * jnp.dot(
        x, wn_ref[...], preferred_element_type=jnp.float32)
    hps_ref[0] = hp[:, :DH]
    hps_ref[1] = hp[:, DH:]


_tc_main = pl.pallas_call(
    _tc_main_body,
    out_shape=(
        jax.ShapeDtypeStruct((N, D), jnp.float32),
        jax.ShapeDtypeStruct((NC, N, DH), jnp.float32),
    ),
)


_AGG_ROWS = ERP // NS
_AHALF = _AGG_ROWS // 2
_NBUF = 8


def _sc_agg_body(hps, col2d, row2d, q, cidx_v, ridx_v, rows_v, zblk_v,
                 acc_sh, gsem, ssem):
    c = lax.axis_index("c")
    s = lax.axis_index("s")

    z16 = jnp.zeros((16,), jnp.float32)
    for r in range(16):
        for cc in range(DH // 16):
            zblk_v[r, pl.ds(cc * 16, 16)] = z16

    def _zero(k, carry):
        pltpu.sync_copy(zblk_v, acc_sh.at[pl.ds(s * 640 + k * 16, 16), :])
        return carry

    lax.fori_loop(0, 40, _zero, 0, unroll=False)

    b0 = s * _AGG_ROWS
    plsc.subcore_barrier()

    hpc = hps.at[c]

    for h in range(2):
        pltpu.sync_copy(col2d.at[pl.ds(b0 + h * _AHALF, _AHALF), :], cidx_v)
        pltpu.sync_copy(row2d.at[pl.ds(b0 + h * _AHALF, _AHALF), :], ridx_v)

        for j0 in range(_NBUF - 1):
            pltpu.async_copy(hpc.at[cidx_v.at[j0]], rows_v.at[j0], gsem)

        def _agg8(jj, carry):
            for b in range(_NBUF):
                j = jj * _NBUF + b
                pltpu.make_async_copy(
                    hpc.at[cidx_v.at[j]], rows_v.at[b], gsem).wait()
                pltpu.async_copy(
                    rows_v.at[b], acc_sh.at[ridx_v.at[j]], ssem, add=True)

                @pl.when(j >= 1)
                def _():
                    pltpu.make_async_copy(
                        rows_v.at[b], acc_sh.at[ridx_v.at[j]], ssem).wait()

                @pl.when(j + _NBUF - 1 < _AHALF)
                def _():
                    pltpu.async_copy(
                        hpc.at[cidx_v.at[j + _NBUF - 1]],
                        rows_v.at[(b + _NBUF - 1) % _NBUF], gsem)
            return carry

        lax.fori_loop(0, _AHALF // _NBUF, _agg8, 0, unroll=False)

        pltpu.make_async_copy(
            rows_v.at[0], acc_sh.at[ridx_v.at[0]], ssem).wait()

    plsc.subcore_barrier()

    pltpu.sync_copy(acc_sh.at[pl.ds(s * 640, 640), :],
                    q.at[pl.ds(c * NPAD + s * 640, 640), :])


_sc_agg = functools.partial(
    pl.kernel,
    out_type=jax.ShapeDtypeStruct((NC * NPAD, DH), jnp.float32),
    mesh=_MESH,
    compiler_params=pltpu.CompilerParams(use_tc_tiling_on_sc=False),
    scratch_types=[
        pltpu.VMEM((_AHALF, CH), jnp.int32),
        pltpu.VMEM((_AHALF, CH), jnp.int32),
        pltpu.VMEM((_NBUF, CH, DH), jnp.float32),
        pltpu.VMEM((16, DH), jnp.float32),
        pltpu.VMEM_SHARED((NPAD, DH), jnp.float32),
        pltpu.SemaphoreType.DMA,
        pltpu.SemaphoreType.DMA,
    ],
)(_sc_agg_body)


def _tc_comb_body(self_ref, dis_ref, q0_ref, q1_ref, o_ref):
    acc = jnp.concatenate([q0_ref[...], q1_ref[...]], axis=1)
    o_ref[...] = self_ref[...] + dis_ref[...] * acc


_tc_comb = pl.pallas_call(
    _tc_comb_body,
    out_shape=jax.ShapeDtypeStruct((N, D), jnp.float32),
)


def kernel(x, edge_index, self_weight, neighbor_weight, bias):
    npad_edges = ERP * D - E
    row_pad = jnp.concatenate(
        [edge_index[0], jnp.full((npad_edges,), NPAD - 1, jnp.int32)])
    col_pad = jnp.concatenate(
        [edge_index[1], jnp.zeros((npad_edges,), jnp.int32)])
    row2d = row_pad.reshape(ERP, D)
    col2d = col_pad.reshape(ERP, D)
    dis128 = _sc_deg(row2d)
    dis_n = dis128[:N]
    selfs, hps = _tc_main(x, self_weight, neighbor_weight,
                          bias.reshape(1, D), dis_n)
    q = _sc_agg(hps, col2d, row2d)
    return _tc_comb(selfs, dis_n, q[:N], q[NPAD:NPAD + N])


# speedup vs baseline: 1.0030x; 1.0030x over previous
import functools

import jax
import jax.numpy as jnp
from jax import lax
from jax.experimental import pallas as pl
from jax.experimental.pallas import tpu as pltpu
from jax.experimental.pallas import tpu_sc as plsc

N = 10000
E = 320000
D = 128
DH = D // 2
NPAD = 10240
NC = 2
NS = 16
ER = E // D
ERP = 2560
CH = 128

_MESH = plsc.VectorSubcoreMesh(
    core_axis_name="c", subcore_axis_name="s", num_cores=NC, num_subcores=NS)

_DEG_ROWS = ERP // NS
_NODES_PER_SUB = NPAD // (NC * NS)


def _sc_deg_body(row2d, dis128, idx_v, ones_v, zro_v, dstage_v, drows_v,
                 deg_sh, sem):
    c = lax.axis_index("c")
    s = lax.axis_index("s")

    z16 = jnp.zeros((16,), jnp.float32)
    for k in range(640 // 16):
        zro_v[pl.ds(k * 16, 16)] = z16
    o16 = jnp.ones((16,), jnp.float32)
    for k in range(CH // 16):
        ones_v[pl.ds(k * 16, 16)] = o16

    pltpu.sync_copy(zro_v, deg_sh.at[pl.ds(s * 640, 640)])

    pltpu.sync_copy(row2d.at[pl.ds(s * _DEG_ROWS, _DEG_ROWS), :], idx_v)

    plsc.subcore_barrier()

    def _scatter(j, carry):
        pltpu.sync_copy(ones_v, deg_sh.at[idx_v.at[j]], add=True)
        return carry

    lax.fori_loop(0, _DEG_ROWS, _scatter, 0, unroll=False)

    plsc.subcore_barrier()

    nbase = (c * NS + s) * _NODES_PER_SUB
    pltpu.sync_copy(deg_sh.at[pl.ds(nbase, _NODES_PER_SUB)], dstage_v)

    for v in range(_NODES_PER_SUB // 16):
        d = dstage_v[pl.ds(v * 16, 16)]
        i = lax.bitcast_convert_type(d, jnp.int32)
        i = jnp.int32(0x5F3759DF) - lax.shift_right_logical(i, 1)
        y = lax.bitcast_convert_type(i, jnp.float32)
        half = d * jnp.float32(0.5)
        for _ in range(3):
            y = y * (jnp.float32(1.5) - half * y * y)
        dstage_v[pl.ds(v * 16, 16)] = y

    def _splat(g, carry):
        v = dstage_v[pl.ds(g * 16, 16)]
        for r in range(16):
            v16 = lax.broadcast(v[r], (16,))
            for cc in range(D // 16):
                drows_v[r, pl.ds(cc * 16, 16)] = v16
        pltpu.sync_copy(drows_v, dis128.at[pl.ds(nbase + g * 16, 16), :])
        return carry

    lax.fori_loop(0, _NODES_PER_SUB // 16, _splat, 0, unroll=False)


_sc_deg = functools.partial(
    pl.kernel,
    out_type=jax.ShapeDtypeStruct((NPAD, D), jnp.float32),
    mesh=_MESH,
    scratch_types=[
        pltpu.VMEM((_DEG_ROWS, CH), jnp.int32),
        pltpu.VMEM((CH,), jnp.float32),
        pltpu.VMEM((640,), jnp.float32),
        pltpu.VMEM((_NODES_PER_SUB,), jnp.float32),
        pltpu.VMEM((16, D), jnp.float32),
        pltpu.VMEM_SHARED((NPAD,), jnp.float32),
        pltpu.SemaphoreType.DMA,
    ],
)(_sc_deg_body)


def _tc_main_body(x_ref, ws_ref, wn_ref, b_ref, dis_ref, self_ref, hps_ref):
    x = x_ref[...]
    self_ref[...] = (
        jnp.dot(x, ws_ref[...], preferred_element_type=jnp.float32)
        + b_ref[...])
    hp = dis_ref[...] * jnp.dot(
        x, wn_ref[...], preferred_element_type=jnp.float32)
    hps_ref[0] = hp[:, :DH]
    hps_ref[1] = hp[:, DH:]


_tc_main = pl.pallas_call(
    _tc_main_body,
    out_shape=(
        jax.ShapeDtypeStruct((N, D), jnp.float32),
        jax.ShapeDtypeStruct((NC, N, DH), jnp.float32),
    ),
)


_AGG_ROWS = ERP // NS
_AHALF = _AGG_ROWS // 2
_NBUF = 8
_G = 3
_L = _NBUF - _G


def _sc_agg_body(hps, col2d, row2d, q, cidx_v, ridx_v, rows_v, zblk_v,
                 acc_sh, gsem, ssem):
    c = lax.axis_index("c")
    s = lax.axis_index("s")

    z16 = jnp.zeros((16,), jnp.float32)
    for r in range(16):
        for cc in range(DH // 16):
            zblk_v[r, pl.ds(cc * 16, 16)] = z16

    def _zero(k, carry):
        pltpu.sync_copy(zblk_v, acc_sh.at[pl.ds(s * 640 + k * 16, 16), :])
        return carry

    lax.fori_loop(0, 40, _zero, 0, unroll=False)

    b0 = s * _AGG_ROWS
    plsc.subcore_barrier()

    hpc = hps.at[c]

    for h in range(2):
        pltpu.sync_copy(col2d.at[pl.ds(b0 + h * _AHALF, _AHALF), :], cidx_v)
        pltpu.sync_copy(row2d.at[pl.ds(b0 + h * _AHALF, _AHALF), :], ridx_v)

        for j0 in range(_G):
            pltpu.async_copy(hpc.at[cidx_v.at[j0]], rows_v.at[j0], gsem)

        def _agg8(jj, carry):
            for b in range(_NBUF):
                j = jj * _NBUF + b
                pltpu.make_async_copy(
                    hpc.at[cidx_v.at[j]], rows_v.at[b], gsem).wait()
                pltpu.async_copy(
                    rows_v.at[b], acc_sh.at[ridx_v.at[j]], ssem, add=True)

                @pl.when(j >= _L)
                def _():
                    pltpu.make_async_copy(
                        rows_v.at[b], acc_sh.at[ridx_v.at[j]], ssem).wait()

                @pl.when(j + _G < _AHALF)
                def _():
                    pltpu.async_copy(
                        hpc.at[cidx_v.at[j + _G]],
                        rows_v.at[(b + _G) % _NBUF], gsem)
            return carry

        lax.fori_loop(0, _AHALF // _NBUF, _agg8, 0, unroll=False)

        for _ in range(_L):
            pltpu.make_async_copy(
                rows_v.at[0], acc_sh.at[ridx_v.at[0]], ssem).wait()

    plsc.subcore_barrier()

    pltpu.sync_copy(acc_sh.at[pl.ds(s * 640, 640), :],
                    q.at[pl.ds(c * NPAD + s * 640, 640), :])


_sc_agg = functools.partial(
    pl.kernel,
    out_type=jax.ShapeDtypeStruct((NC * NPAD, DH), jnp.float32),
    mesh=_MESH,
    compiler_params=pltpu.CompilerParams(use_tc_tiling_on_sc=False),
    scratch_types=[
        pltpu.VMEM((_AHALF, CH), jnp.int32),
        pltpu.VMEM((_AHALF, CH), jnp.int32),
        pltpu.VMEM((_NBUF, CH, DH), jnp.float32),
        pltpu.VMEM((16, DH), jnp.float32),
        pltpu.VMEM_SHARED((NPAD, DH), jnp.float32),
        pltpu.SemaphoreType.DMA,
        pltpu.SemaphoreType.DMA,
    ],
)(_sc_agg_body)


def _tc_comb_body(self_ref, dis_ref, q0_ref, q1_ref, o_ref):
    acc = jnp.concatenate([q0_ref[...], q1_ref[...]], axis=1)
    o_ref[...] = self_ref[...] + dis_ref[...] * acc


_tc_comb = pl.pallas_call(
    _tc_comb_body,
    out_shape=jax.ShapeDtypeStruct((N, D), jnp.float32),
)


def kernel(x, edge_index, self_weight, neighbor_weight, bias):
    npad_edges = ERP * D - E
    row_pad = jnp.concatenate(
        [edge_index[0], jnp.full((npad_edges,), NPAD - 1, jnp.int32)])
    col_pad = jnp.concatenate(
        [edge_index[1], jnp.zeros((npad_edges,), jnp.int32)])
    row2d = row_pad.reshape(ERP, D)
    col2d = col_pad.reshape(ERP, D)
    dis128 = _sc_deg(row2d)
    dis_n = dis128[:N]
    selfs, hps = _tc_main(x, self_weight, neighbor_weight,
                          bias.reshape(1, D), dis_n)
    q = _sc_agg(hps, col2d, row2d)
    return _tc_comb(selfs, dis_n, q[:N], q[NPAD:NPAD + N])


# speedup vs baseline: 1.6168x; 1.6119x over previous
import functools

import jax
import jax.numpy as jnp
from jax import lax
from jax.experimental import pallas as pl
from jax.experimental.pallas import tpu as pltpu
from jax.experimental.pallas import tpu_sc as plsc

N = 10000
E = 320000
D = 128
DH = D // 2
NPAD = 10240
NC = 2
NS = 16
ER = E // D
ERP = 2560
CH = 128

_MESH = plsc.VectorSubcoreMesh(
    core_axis_name="c", subcore_axis_name="s", num_cores=NC, num_subcores=NS)

_DEG_ROWS = ERP // NS
_NODES_PER_SUB = NPAD // (NC * NS)


def _sc_deg_body(row2d, dis128, idx_v, ones_v, zro_v, dstage_v, drows_v,
                 deg_sh, sem):
    c = lax.axis_index("c")
    s = lax.axis_index("s")

    z16 = jnp.zeros((16,), jnp.float32)
    for k in range(640 // 16):
        zro_v[pl.ds(k * 16, 16)] = z16
    o16 = jnp.ones((16,), jnp.float32)
    for k in range(CH // 16):
        ones_v[pl.ds(k * 16, 16)] = o16

    pltpu.sync_copy(zro_v, deg_sh.at[pl.ds(s * 640, 640)])

    pltpu.sync_copy(row2d.at[pl.ds(s * _DEG_ROWS, _DEG_ROWS), :], idx_v)

    plsc.subcore_barrier()

    def _scatter(j, carry):
        pltpu.sync_copy(ones_v, deg_sh.at[idx_v.at[j]], add=True)
        return carry

    lax.fori_loop(0, _DEG_ROWS, _scatter, 0, unroll=False)

    plsc.subcore_barrier()

    nbase = (c * NS + s) * _NODES_PER_SUB
    pltpu.sync_copy(deg_sh.at[pl.ds(nbase, _NODES_PER_SUB)], dstage_v)

    for v in range(_NODES_PER_SUB // 16):
        d = dstage_v[pl.ds(v * 16, 16)]
        i = lax.bitcast_convert_type(d, jnp.int32)
        i = jnp.int32(0x5F3759DF) - lax.shift_right_logical(i, 1)
        y = lax.bitcast_convert_type(i, jnp.float32)
        half = d * jnp.float32(0.5)
        for _ in range(3):
            y = y * (jnp.float32(1.5) - half * y * y)
        dstage_v[pl.ds(v * 16, 16)] = y

    def _splat(g, carry):
        v = dstage_v[pl.ds(g * 16, 16)]
        for r in range(16):
            v16 = lax.broadcast(v[r], (16,))
            for cc in range(D // 16):
                drows_v[r, pl.ds(cc * 16, 16)] = v16
        pltpu.sync_copy(drows_v, dis128.at[pl.ds(nbase + g * 16, 16), :])
        return carry

    lax.fori_loop(0, _NODES_PER_SUB // 16, _splat, 0, unroll=False)


_sc_deg = functools.partial(
    pl.kernel,
    out_type=jax.ShapeDtypeStruct((NPAD, D), jnp.float32),
    mesh=_MESH,
    scratch_types=[
        pltpu.VMEM((_DEG_ROWS, CH), jnp.int32),
        pltpu.VMEM((CH,), jnp.float32),
        pltpu.VMEM((640,), jnp.float32),
        pltpu.VMEM((_NODES_PER_SUB,), jnp.float32),
        pltpu.VMEM((16, D), jnp.float32),
        pltpu.VMEM_SHARED((NPAD,), jnp.float32),
        pltpu.SemaphoreType.DMA,
    ],
)(_sc_deg_body)


def _tc_main_body(x_ref, ws_ref, wn_ref, b_ref, dis_ref, self_ref, hps_ref):
    x = x_ref[...]
    self_ref[...] = (
        jnp.dot(x, ws_ref[...], preferred_element_type=jnp.float32)
        + b_ref[...])
    hp = dis_ref[...] * jnp.dot(
        x, wn_ref[...], preferred_element_type=jnp.float32)
    hps_ref[0] = hp[:, :DH].astype(jnp.bfloat16)
    hps_ref[1] = hp[:, DH:].astype(jnp.bfloat16)


_tc_main = pl.pallas_call(
    _tc_main_body,
    out_shape=(
        jax.ShapeDtypeStruct((N, D), jnp.float32),
        jax.ShapeDtypeStruct((NC, N, DH), jnp.bfloat16),
    ),
)


_AGG_ROWS = ERP // NS
_AHALF = _AGG_ROWS // 2
_NBUF = 8
_G = 3
_L = _NBUF - _G


def _sc_agg_body(hps, col2d, row2d, q, cidx_v, ridx_v, rows_v, zblk_v,
                 acc_sh, gsem, ssem):
    c = lax.axis_index("c")
    s = lax.axis_index("s")

    z32 = jnp.zeros((32,), jnp.bfloat16)
    for r in range(16):
        for cc in range(DH // 32):
            zblk_v[r, pl.ds(cc * 32, 32)] = z32

    def _zero(k, carry):
        pltpu.sync_copy(zblk_v, acc_sh.at[pl.ds(s * 640 + k * 16, 16), :])
        return carry

    lax.fori_loop(0, 40, _zero, 0, unroll=False)

    b0 = s * _AGG_ROWS
    pltpu.sync_copy(col2d.at[pl.ds(b0, _AGG_ROWS), :], cidx_v)
    pltpu.sync_copy(row2d.at[pl.ds(b0, _AGG_ROWS), :], ridx_v)
    plsc.subcore_barrier()

    hpc = hps.at[c]

    for j0 in range(_G):
        pltpu.async_copy(hpc.at[cidx_v.at[j0]], rows_v.at[j0], gsem)

    def _agg8(jj, carry):
        for b in range(_NBUF):
            j = jj * _NBUF + b
            pltpu.make_async_copy(
                hpc.at[cidx_v.at[j]], rows_v.at[b], gsem).wait()
            pltpu.async_copy(
                rows_v.at[b], acc_sh.at[ridx_v.at[j]], ssem, add=True)

            @pl.when(j >= _L)
            def _():
                pltpu.make_async_copy(
                    rows_v.at[b], acc_sh.at[ridx_v.at[j]], ssem).wait()

            @pl.when(j + _G < _AGG_ROWS)
            def _():
                pltpu.async_copy(
                    hpc.at[cidx_v.at[j + _G]],
                    rows_v.at[(b + _G) % _NBUF], gsem)
        return carry

    lax.fori_loop(0, _AGG_ROWS // _NBUF, _agg8, 0, unroll=False)

    for _ in range(_L):
        pltpu.make_async_copy(
            rows_v.at[0], acc_sh.at[ridx_v.at[0]], ssem).wait()

    plsc.subcore_barrier()

    pltpu.sync_copy(acc_sh.at[pl.ds(s * 640, 640), :],
                    q.at[pl.ds(c * NPAD + s * 640, 640), :])


_sc_agg = functools.partial(
    pl.kernel,
    out_type=jax.ShapeDtypeStruct((NC * NPAD, DH), jnp.bfloat16),
    mesh=_MESH,
    compiler_params=pltpu.CompilerParams(use_tc_tiling_on_sc=False),
    scratch_types=[
        pltpu.VMEM((_AGG_ROWS, CH), jnp.int32),
        pltpu.VMEM((_AGG_ROWS, CH), jnp.int32),
        pltpu.VMEM((_NBUF, CH, DH), jnp.bfloat16),
        pltpu.VMEM((16, DH), jnp.bfloat16),
        pltpu.VMEM_SHARED((NPAD, DH), jnp.bfloat16),
        pltpu.SemaphoreType.DMA,
        pltpu.SemaphoreType.DMA,
    ],
)(_sc_agg_body)


def _tc_comb_body(self_ref, dis_ref, q0_ref, q1_ref, o_ref):
    acc = jnp.concatenate([q0_ref[...], q1_ref[...]], axis=1)
    o_ref[...] = self_ref[...] + dis_ref[...] * acc.astype(jnp.float32)


_tc_comb = pl.pallas_call(
    _tc_comb_body,
    out_shape=jax.ShapeDtypeStruct((N, D), jnp.float32),
)


def kernel(x, edge_index, self_weight, neighbor_weight, bias):
    npad_edges = ERP * D - E
    row_pad = jnp.concatenate(
        [edge_index[0], jnp.full((npad_edges,), NPAD - 1, jnp.int32)])
    col_pad = jnp.concatenate(
        [edge_index[1], jnp.zeros((npad_edges,), jnp.int32)])
    row2d = row_pad.reshape(ERP, D)
    col2d = col_pad.reshape(ERP, D)
    dis128 = _sc_deg(row2d)
    dis_n = dis128[:N]
    selfs, hps = _tc_main(x, self_weight, neighbor_weight,
                          bias.reshape(1, D), dis_n)
    q = _sc_agg(hps, col2d, row2d)
    return _tc_comb(selfs, dis_n, q[:N], q[NPAD:NPAD + N])


# speedup vs baseline: 1.6299x; 1.0081x over previous
import functools

import jax
import jax.numpy as jnp
from jax import lax
from jax.experimental import pallas as pl
from jax.experimental.pallas import tpu as pltpu
from jax.experimental.pallas import tpu_sc as plsc

N = 10000
E = 320000
D = 128
DH = D // 2
NPAD = 10240
NC = 2
NS = 16
ER = E // D
ERP = 2560
CH = 128

_MESH = plsc.VectorSubcoreMesh(
    core_axis_name="c", subcore_axis_name="s", num_cores=NC, num_subcores=NS)

_DEG_ROWS = ERP // NS
_NODES_PER_SUB = NPAD // (NC * NS)


def _sc_deg_body(row2d, dis128, dis1d, idx_v, ones_v, zro_v, dstage_v,
                 drows_v, deg_sh, sem):
    c = lax.axis_index("c")
    s = lax.axis_index("s")

    z16 = jnp.zeros((16,), jnp.float32)
    for k in range(640 // 16):
        zro_v[pl.ds(k * 16, 16)] = z16
    o16 = jnp.ones((16,), jnp.float32)
    for k in range(CH // 16):
        ones_v[pl.ds(k * 16, 16)] = o16

    pltpu.sync_copy(zro_v, deg_sh.at[pl.ds(s * 640, 640)])

    pltpu.sync_copy(row2d.at[pl.ds(s * _DEG_ROWS, _DEG_ROWS), :], idx_v)

    plsc.subcore_barrier()

    def _scatter(j, carry):
        pltpu.async_copy(ones_v, deg_sh.at[idx_v.at[j]], sem, add=True)

        @pl.when(j >= 4)
        def _():
            pltpu.make_async_copy(ones_v, deg_sh.at[idx_v.at[j]], sem).wait()
        return carry

    lax.fori_loop(0, _DEG_ROWS, _scatter, 0, unroll=False)
    for _ in range(4):
        pltpu.make_async_copy(ones_v, deg_sh.at[idx_v.at[0]], sem).wait()

    plsc.subcore_barrier()

    nbase = (c * NS + s) * _NODES_PER_SUB
    pltpu.sync_copy(deg_sh.at[pl.ds(nbase, _NODES_PER_SUB)], dstage_v)

    for v in range(_NODES_PER_SUB // 16):
        d = dstage_v[pl.ds(v * 16, 16)]
        i = lax.bitcast_convert_type(d, jnp.int32)
        i = jnp.int32(0x5F3759DF) - lax.shift_right_logical(i, 1)
        y = lax.bitcast_convert_type(i, jnp.float32)
        half = d * jnp.float32(0.5)
        for _ in range(3):
            y = y * (jnp.float32(1.5) - half * y * y)
        dstage_v[pl.ds(v * 16, 16)] = y

    def _splat(g, carry):
        v = dstage_v[pl.ds(g * 16, 16)]
        for r in range(16):
            v16 = lax.broadcast(v[r], (16,))
            for cc in range(D // 16):
                drows_v[r, pl.ds(cc * 16, 16)] = v16
        pltpu.sync_copy(drows_v, dis128.at[pl.ds(nbase + g * 16, 16), :])
        return carry

    lax.fori_loop(0, _NODES_PER_SUB // 16, _splat, 0, unroll=False)

    pltpu.sync_copy(dstage_v, dis1d.at[pl.ds(nbase, _NODES_PER_SUB)])


_sc_deg = functools.partial(
    pl.kernel,
    out_type=(jax.ShapeDtypeStruct((NPAD, D), jnp.float32),
              jax.ShapeDtypeStruct((NPAD,), jnp.float32)),
    mesh=_MESH,
    scratch_types=[
        pltpu.VMEM((_DEG_ROWS, CH), jnp.int32),
        pltpu.VMEM((CH,), jnp.float32),
        pltpu.VMEM((640,), jnp.float32),
        pltpu.VMEM((_NODES_PER_SUB,), jnp.float32),
        pltpu.VMEM((16, D), jnp.float32),
        pltpu.VMEM_SHARED((NPAD,), jnp.float32),
        pltpu.SemaphoreType.DMA,
    ],
)(_sc_deg_body)


def _tc_main_body(x_ref, ws_ref, wn_ref, b_ref, dis_ref, self_ref, hps_ref):
    x = x_ref[...]
    self_ref[...] = (
        jnp.dot(x, ws_ref[...], preferred_element_type=jnp.float32)
        + b_ref[...])
    hp = dis_ref[...] * jnp.dot(
        x, wn_ref[...], preferred_element_type=jnp.float32)
    hps_ref[0] = hp[:, :DH].astype(jnp.bfloat16)
    hps_ref[1] = hp[:, DH:].astype(jnp.bfloat16)


_tc_main = pl.pallas_call(
    _tc_main_body,
    out_shape=(
        jax.ShapeDtypeStruct((N, D), jnp.float32),
        jax.ShapeDtypeStruct((NC, N, DH), jnp.bfloat16),
    ),
)


_AGG_ROWS = ERP // NS
_AHALF = _AGG_ROWS // 2
_NBUF = 8
_G = 3
_L = _NBUF - _G


def _sc_agg_body(hps, col2d, row2d, q, cidx_v, ridx_v, rows_v, zblk_v,
                 acc_sh, gsem, ssem):
    c = lax.axis_index("c")
    s = lax.axis_index("s")

    z32 = jnp.zeros((32,), jnp.bfloat16)
    for r in range(16):
        for cc in range(DH // 32):
            zblk_v[r, pl.ds(cc * 32, 32)] = z32

    def _zero(k, carry):
        pltpu.sync_copy(zblk_v, acc_sh.at[pl.ds(s * 640 + k * 16, 16), :])
        return carry

    lax.fori_loop(0, 40, _zero, 0, unroll=False)

    b0 = s * _AGG_ROWS
    pltpu.sync_copy(col2d.at[pl.ds(b0, _AGG_ROWS), :], cidx_v)
    pltpu.sync_copy(row2d.at[pl.ds(b0, _AGG_ROWS), :], ridx_v)
    plsc.subcore_barrier()

    hpc = hps.at[c]

    for j0 in range(_G):
        pltpu.async_copy(hpc.at[cidx_v.at[j0]], rows_v.at[j0], gsem)

    def _agg8(jj, carry):
        for b in range(_NBUF):
            j = jj * _NBUF + b
            pltpu.make_async_copy(
                hpc.at[cidx_v.at[j]], rows_v.at[b], gsem).wait()
            pltpu.async_copy(
                rows_v.at[b], acc_sh.at[ridx_v.at[j]], ssem, add=True)

            @pl.when(j >= _L)
            def _():
                pltpu.make_async_copy(
                    rows_v.at[b], acc_sh.at[ridx_v.at[j]], ssem).wait()

            @pl.when(j + _G < _AGG_ROWS)
            def _():
                pltpu.async_copy(
                    hpc.at[cidx_v.at[j + _G]],
                    rows_v.at[(b + _G) % _NBUF], gsem)
        return carry

    lax.fori_loop(0, _AGG_ROWS // _NBUF, _agg8, 0, unroll=False)

    for _ in range(_L):
        pltpu.make_async_copy(
            rows_v.at[0], acc_sh.at[ridx_v.at[0]], ssem).wait()

    plsc.subcore_barrier()

    pltpu.sync_copy(acc_sh.at[pl.ds(s * 640, 640), :],
                    q.at[pl.ds(c * NPAD + s * 640, 640), :])


_sc_agg = functools.partial(
    pl.kernel,
    out_type=jax.ShapeDtypeStruct((NC * NPAD, DH), jnp.bfloat16),
    mesh=_MESH,
    compiler_params=pltpu.CompilerParams(use_tc_tiling_on_sc=False),
    scratch_types=[
        pltpu.VMEM((_AGG_ROWS, CH), jnp.int32),
        pltpu.VMEM((_AGG_ROWS, CH), jnp.int32),
        pltpu.VMEM((_NBUF, CH, DH), jnp.bfloat16),
        pltpu.VMEM((16, DH), jnp.bfloat16),
        pltpu.VMEM_SHARED((NPAD, DH), jnp.bfloat16),
        pltpu.SemaphoreType.DMA,
        pltpu.SemaphoreType.DMA,
    ],
)(_sc_agg_body)


def _tc_comb_body(self_ref, dis_ref, q0_ref, q1_ref, o_ref):
    acc = jnp.concatenate([q0_ref[...], q1_ref[...]], axis=1)
    o_ref[...] = self_ref[...] + dis_ref[...] * acc.astype(jnp.float32)


_tc_comb = pl.pallas_call(
    _tc_comb_body,
    out_shape=jax.ShapeDtypeStruct((N, D), jnp.float32),
)


def kernel(x, edge_index, self_weight, neighbor_weight, bias):
    npad_edges = ERP * D - E
    row_pad = jnp.concatenate(
        [edge_index[0], jnp.full((npad_edges,), NPAD - 1, jnp.int32)])
    col_pad = jnp.concatenate(
        [edge_index[1], jnp.zeros((npad_edges,), jnp.int32)])
    row2d = row_pad.reshape(ERP, D)
    col2d = col_pad.reshape(ERP, D)
    dis128, dis1d = _sc_deg(row2d)
    dis_n = dis128[:N]
    selfs, hps = _tc_main(x, self_weight, neighbor_weight,
                          bias.reshape(1, D), dis_n)
    q = _sc_agg(hps, col2d, row2d)
    return _tc_comb(selfs, dis_n, q[:N], q[NPAD:NPAD + N])


# speedup vs baseline: 1.6683x; 1.0235x over previous
import functools

import jax
import jax.numpy as jnp
from jax import lax
from jax.experimental import pallas as pl
from jax.experimental.pallas import tpu as pltpu
from jax.experimental.pallas import tpu_sc as plsc

N = 10000
E = 320000
D = 128
DH = D // 2
NPAD = 10240
NC = 2
NS = 16
ER = E // D
ERP = 2560
CH = 128

_MESH = plsc.VectorSubcoreMesh(
    core_axis_name="c", subcore_axis_name="s", num_cores=NC, num_subcores=NS)

_DEG_ROWS = ERP // NS
_NODES_PER_SUB = NPAD // (NC * NS)


def _sc_deg_body(row2d, dis128, idx_v, ones_v, zro_v, dstage_v,
                 drows_v, deg_sh, sem):
    c = lax.axis_index("c")
    s = lax.axis_index("s")

    z16 = jnp.zeros((16,), jnp.float32)
    for k in range(640 // 16):
        zro_v[pl.ds(k * 16, 16)] = z16
    o16 = jnp.ones((16,), jnp.float32)
    for k in range(CH // 16):
        ones_v[pl.ds(k * 16, 16)] = o16

    pltpu.sync_copy(zro_v, deg_sh.at[pl.ds(s * 640, 640)])

    pltpu.sync_copy(row2d.at[pl.ds(s * _DEG_ROWS, _DEG_ROWS), :], idx_v)

    plsc.subcore_barrier()

    def _scatter(j, carry):
        pltpu.async_copy(ones_v, deg_sh.at[idx_v.at[j]], sem, add=True)

        @pl.when(j >= 4)
        def _():
            pltpu.make_async_copy(ones_v, deg_sh.at[idx_v.at[j]], sem).wait()
        return carry

    lax.fori_loop(0, _DEG_ROWS, _scatter, 0, unroll=False)
    for _ in range(4):
        pltpu.make_async_copy(ones_v, deg_sh.at[idx_v.at[0]], sem).wait()

    plsc.subcore_barrier()

    nbase = (c * NS + s) * _NODES_PER_SUB
    pltpu.sync_copy(deg_sh.at[pl.ds(nbase, _NODES_PER_SUB)], dstage_v)

    for v in range(_NODES_PER_SUB // 16):
        d = dstage_v[pl.ds(v * 16, 16)]
        i = lax.bitcast_convert_type(d, jnp.int32)
        i = jnp.int32(0x5F3759DF) - lax.shift_right_logical(i, 1)
        y = lax.bitcast_convert_type(i, jnp.float32)
        half = d * jnp.float32(0.5)
        for _ in range(3):
            y = y * (jnp.float32(1.5) - half * y * y)
        dstage_v[pl.ds(v * 16, 16)] = y

    def _splat(g, carry):
        v = dstage_v[pl.ds(g * 16, 16)]
        for r in range(16):
            v16 = lax.broadcast(v[r], (16,))
            for cc in range(D // 16):
                drows_v[r, pl.ds(cc * 16, 16)] = v16
        pltpu.sync_copy(drows_v, dis128.at[pl.ds(nbase + g * 16, 16), :])
        return carry

    lax.fori_loop(0, _NODES_PER_SUB // 16, _splat, 0, unroll=False)


_sc_deg = functools.partial(
    pl.kernel,
    out_type=jax.ShapeDtypeStruct((NPAD, D), jnp.float32),
    mesh=_MESH,
    scratch_types=[
        pltpu.VMEM((_DEG_ROWS, CH), jnp.int32),
        pltpu.VMEM((CH,), jnp.float32),
        pltpu.VMEM((640,), jnp.float32),
        pltpu.VMEM((_NODES_PER_SUB,), jnp.float32),
        pltpu.VMEM((16, D), jnp.float32),
        pltpu.VMEM_SHARED((NPAD,), jnp.float32),
        pltpu.SemaphoreType.DMA,
    ],
)(_sc_deg_body)


def _tc_main_body(x_ref, ws_ref, wn_ref, b_ref, dis_ref, self_ref, hps_ref):
    x = x_ref[...]
    self_ref[...] = (
        jnp.dot(x, ws_ref[...], preferred_element_type=jnp.float32)
        + b_ref[...])
    hp = dis_ref[...] * jnp.dot(
        x, wn_ref[...], preferred_element_type=jnp.float32)
    hps_ref[0] = hp[:, :DH].astype(jnp.bfloat16)
    hps_ref[1] = hp[:, DH:].astype(jnp.bfloat16)


_tc_main = pl.pallas_call(
    _tc_main_body,
    out_shape=(
        jax.ShapeDtypeStruct((N, D), jnp.float32),
        jax.ShapeDtypeStruct((NC, N, DH), jnp.bfloat16),
    ),
)


_AGG_ROWS = ERP // NS
_AHALF = _AGG_ROWS // 2
_NBUF = 8
_G = 3
_L = _NBUF - _G


def _sc_agg_body(hps, col2d, row2d, q, cidx_v, ridx_v, rows_v, zblk_v,
                 acc_sh, gsem, ssem):
    c = lax.axis_index("c")
    s = lax.axis_index("s")

    b0 = s * _AGG_ROWS
    pltpu.async_copy(col2d.at[pl.ds(b0, _AGG_ROWS), :], cidx_v, gsem)
    pltpu.async_copy(row2d.at[pl.ds(b0, _AGG_ROWS), :], ridx_v, gsem)

    z32 = jnp.zeros((32,), jnp.bfloat16)
    for r in range(128):
        for cc in range(DH // 32):
            zblk_v[r, pl.ds(cc * 32, 32)] = z32

    for k in range(5):
        pltpu.async_copy(
            zblk_v, acc_sh.at[pl.ds(s * 640 + k * 128, 128), :], ssem)
    for k in range(5):
        pltpu.make_async_copy(
            zblk_v, acc_sh.at[pl.ds(s * 640, 128), :], ssem).wait()
    pltpu.make_async_copy(
        col2d.at[pl.ds(b0, _AGG_ROWS), :], cidx_v, gsem).wait()
    pltpu.make_async_copy(
        row2d.at[pl.ds(b0, _AGG_ROWS), :], ridx_v, gsem).wait()
    plsc.subcore_barrier()

    hpc = hps.at[c]

    for j0 in range(_G):
        pltpu.async_copy(hpc.at[cidx_v.at[j0]], rows_v.at[j0], gsem)

    def _agg8(jj, carry):
        for b in range(_NBUF):
            j = jj * _NBUF + b
            pltpu.make_async_copy(
                hpc.at[cidx_v.at[j]], rows_v.at[b], gsem).wait()
            pltpu.async_copy(
                rows_v.at[b], acc_sh.at[ridx_v.at[j]], ssem, add=True)

            @pl.when(j >= _L)
            def _():
                pltpu.make_async_copy(
                    rows_v.at[b], acc_sh.at[ridx_v.at[j]], ssem).wait()

            @pl.when(j + _G < _AGG_ROWS)
            def _():
                pltpu.async_copy(
                    hpc.at[cidx_v.at[j + _G]],
                    rows_v.at[(b + _G) % _NBUF], gsem)
        return carry

    lax.fori_loop(0, _AGG_ROWS // _NBUF, _agg8, 0, unroll=False)

    for _ in range(_L):
        pltpu.make_async_copy(
            rows_v.at[0], acc_sh.at[ridx_v.at[0]], ssem).wait()

    plsc.subcore_barrier()

    pltpu.sync_copy(acc_sh.at[pl.ds(s * 640, 640), :],
                    q.at[pl.ds(c * NPAD + s * 640, 640), :])


_sc_agg = functools.partial(
    pl.kernel,
    out_type=jax.ShapeDtypeStruct((NC * NPAD, DH), jnp.bfloat16),
    mesh=_MESH,
    compiler_params=pltpu.CompilerParams(use_tc_tiling_on_sc=False),
    scratch_types=[
        pltpu.VMEM((_AGG_ROWS, CH), jnp.int32),
        pltpu.VMEM((_AGG_ROWS, CH), jnp.int32),
        pltpu.VMEM((_NBUF, CH, DH), jnp.bfloat16),
        pltpu.VMEM((128, DH), jnp.bfloat16),
        pltpu.VMEM_SHARED((NPAD, DH), jnp.bfloat16),
        pltpu.SemaphoreType.DMA,
        pltpu.SemaphoreType.DMA,
    ],
)(_sc_agg_body)


def _tc_comb_body(self_ref, dis_ref, q0_ref, q1_ref, o_ref):
    acc = jnp.concatenate([q0_ref[...], q1_ref[...]], axis=1)
    o_ref[...] = self_ref[...] + dis_ref[...] * acc.astype(jnp.float32)


_tc_comb = pl.pallas_call(
    _tc_comb_body,
    out_shape=jax.ShapeDtypeStruct((N, D), jnp.float32),
)


def kernel(x, edge_index, self_weight, neighbor_weight, bias):
    npad_edges = ERP * D - E
    row_pad = jnp.concatenate(
        [edge_index[0], jnp.full((npad_edges,), NPAD - 1, jnp.int32)])
    col_pad = jnp.concatenate(
        [edge_index[1], jnp.zeros((npad_edges,), jnp.int32)])
    row2d = row_pad.reshape(ERP, D)
    col2d = col_pad.reshape(ERP, D)
    dis128 = _sc_deg(row2d)
    dis_n = dis128[:N]
    selfs, hps = _tc_main(x, self_weight, neighbor_weight,
                          bias.reshape(1, D), dis_n)
    q = _sc_agg(hps, col2d, row2d)
    return _tc_comb(selfs, dis_n, q[:N], q[NPAD:NPAD + N])


# speedup vs baseline: 1.7561x; 1.0526x over previous
import functools

import jax
import jax.numpy as jnp
from jax import lax
from jax.experimental import pallas as pl
from jax.experimental.pallas import tpu as pltpu
from jax.experimental.pallas import tpu_sc as plsc

N = 10000
E = 320000
D = 128
DH = D // 2
NPAD = 10240
NC = 2
NS = 16
ER = E // D
ERP = 2560
CH = 128

_MESH = plsc.VectorSubcoreMesh(
    core_axis_name="c", subcore_axis_name="s", num_cores=NC, num_subcores=NS)

_DEG_ROWS = ERP // NS
_NODES_PER_SUB = NPAD // (NC * NS)


def _sc_deg_body(row2d, dis128, idx_v, ones_v, zro_v, dstage_v,
                 drows_v, deg_sh, sem):
    c = lax.axis_index("c")
    s = lax.axis_index("s")

    z16 = jnp.zeros((16,), jnp.float32)
    for k in range(640 // 16):
        zro_v[pl.ds(k * 16, 16)] = z16
    o16 = jnp.ones((16,), jnp.float32)
    for k in range(CH // 16):
        ones_v[pl.ds(k * 16, 16)] = o16

    pltpu.sync_copy(zro_v, deg_sh.at[pl.ds(s * 640, 640)])

    pltpu.sync_copy(row2d.at[pl.ds(s * _DEG_ROWS, _DEG_ROWS), :], idx_v)

    plsc.subcore_barrier()

    def _scatter(j, carry):
        pltpu.async_copy(ones_v, deg_sh.at[idx_v.at[j]], sem, add=True)

        @pl.when(j >= 4)
        def _():
            pltpu.make_async_copy(ones_v, deg_sh.at[idx_v.at[j]], sem).wait()
        return carry

    lax.fori_loop(0, _DEG_ROWS, _scatter, 0, unroll=False)
    for _ in range(4):
        pltpu.make_async_copy(ones_v, deg_sh.at[idx_v.at[0]], sem).wait()

    plsc.subcore_barrier()

    nbase = (c * NS + s) * _NODES_PER_SUB
    pltpu.sync_copy(deg_sh.at[pl.ds(nbase, _NODES_PER_SUB)], dstage_v)

    for v in range(_NODES_PER_SUB // 16):
        d = dstage_v[pl.ds(v * 16, 16)]
        i = lax.bitcast_convert_type(d, jnp.int32)
        i = jnp.int32(0x5F3759DF) - lax.shift_right_logical(i, 1)
        y = lax.bitcast_convert_type(i, jnp.float32)
        half = d * jnp.float32(0.5)
        for _ in range(3):
            y = y * (jnp.float32(1.5) - half * y * y)
        dstage_v[pl.ds(v * 16, 16)] = y

    def _splat(g, carry):
        v = dstage_v[pl.ds(g * 16, 16)]
        for r in range(16):
            v16 = lax.broadcast(v[r], (16,))
            for cc in range(D // 16):
                drows_v[r, pl.ds(cc * 16, 16)] = v16
        pltpu.sync_copy(drows_v, dis128.at[pl.ds(nbase + g * 16, 16), :])
        return carry

    lax.fori_loop(0, _NODES_PER_SUB // 16, _splat, 0, unroll=False)


_sc_deg = functools.partial(
    pl.kernel,
    out_type=jax.ShapeDtypeStruct((NPAD, D), jnp.float32),
    mesh=_MESH,
    scratch_types=[
        pltpu.VMEM((_DEG_ROWS, CH), jnp.int32),
        pltpu.VMEM((CH,), jnp.float32),
        pltpu.VMEM((640,), jnp.float32),
        pltpu.VMEM((_NODES_PER_SUB,), jnp.float32),
        pltpu.VMEM((16, D), jnp.float32),
        pltpu.VMEM_SHARED((NPAD,), jnp.float32),
        pltpu.SemaphoreType.DMA,
    ],
)(_sc_deg_body)


def _tc_main_body(x_ref, ws_ref, wn_ref, b_ref, dis_ref, self_ref, hps_ref):
    x = x_ref[...]
    self_ref[...] = (
        jnp.dot(x, ws_ref[...], preferred_element_type=jnp.float32)
        + b_ref[...])
    hp = dis_ref[:N] * jnp.dot(
        x, wn_ref[...], preferred_element_type=jnp.float32)
    hps_ref[0] = hp[:, :DH].astype(jnp.bfloat16)
    hps_ref[1] = hp[:, DH:].astype(jnp.bfloat16)


_tc_main = pl.pallas_call(
    _tc_main_body,
    out_shape=(
        jax.ShapeDtypeStruct((N, D), jnp.float32),
        jax.ShapeDtypeStruct((NC, N, DH), jnp.bfloat16),
    ),
)


_AGG_ROWS = ERP // NS
_AHALF = _AGG_ROWS // 2
_NBUF = 8
_G = 3
_L = _NBUF - _G


def _sc_agg_body(hps, col2d, row2d, q, cidx_v, ridx_v, rows_v, zblk_v,
                 acc_sh, gsem, ssem):
    c = lax.axis_index("c")
    s = lax.axis_index("s")

    b0 = s * _AGG_ROWS
    pltpu.async_copy(col2d.at[pl.ds(b0, _AGG_ROWS), :], cidx_v, gsem)
    pltpu.async_copy(row2d.at[pl.ds(b0, _AGG_ROWS), :], ridx_v, gsem)

    z32 = jnp.zeros((32,), jnp.bfloat16)
    for r in range(128):
        for cc in range(DH // 32):
            zblk_v[r, pl.ds(cc * 32, 32)] = z32

    for k in range(5):
        pltpu.async_copy(
            zblk_v, acc_sh.at[pl.ds(s * 640 + k * 128, 128), :], ssem)
    for k in range(5):
        pltpu.make_async_copy(
            zblk_v, acc_sh.at[pl.ds(s * 640, 128), :], ssem).wait()
    pltpu.make_async_copy(
        col2d.at[pl.ds(b0, _AGG_ROWS), :], cidx_v, gsem).wait()
    pltpu.make_async_copy(
        row2d.at[pl.ds(b0, _AGG_ROWS), :], ridx_v, gsem).wait()
    plsc.subcore_barrier()

    hpc = hps.at[c]

    for j0 in range(_G):
        pltpu.async_copy(hpc.at[cidx_v.at[j0]], rows_v.at[j0], gsem)

    def _agg8(jj, carry):
        for b in range(_NBUF):
            j = jj * _NBUF + b
            pltpu.make_async_copy(
                hpc.at[cidx_v.at[j]], rows_v.at[b], gsem).wait()
            pltpu.async_copy(
                rows_v.at[b], acc_sh.at[ridx_v.at[j]], ssem, add=True)

            @pl.when(j >= _L)
            def _():
                pltpu.make_async_copy(
                    rows_v.at[b], acc_sh.at[ridx_v.at[j]], ssem).wait()

            @pl.when(j + _G < _AGG_ROWS)
            def _():
                pltpu.async_copy(
                    hpc.at[cidx_v.at[j + _G]],
                    rows_v.at[(b + _G) % _NBUF], gsem)
        return carry

    lax.fori_loop(0, _AGG_ROWS // _NBUF, _agg8, 0, unroll=False)

    for _ in range(_L):
        pltpu.make_async_copy(
            rows_v.at[0], acc_sh.at[ridx_v.at[0]], ssem).wait()

    plsc.subcore_barrier()

    pltpu.sync_copy(acc_sh.at[pl.ds(s * 640, 640), :],
                    q.at[pl.ds(c * NPAD + s * 640, 640), :])


_sc_agg = functools.partial(
    pl.kernel,
    out_type=jax.ShapeDtypeStruct((NC * NPAD, DH), jnp.bfloat16),
    mesh=_MESH,
    compiler_params=pltpu.CompilerParams(use_tc_tiling_on_sc=False),
    scratch_types=[
        pltpu.VMEM((_AGG_ROWS, CH), jnp.int32),
        pltpu.VMEM((_AGG_ROWS, CH), jnp.int32),
        pltpu.VMEM((_NBUF, CH, DH), jnp.bfloat16),
        pltpu.VMEM((128, DH), jnp.bfloat16),
        pltpu.VMEM_SHARED((NPAD, DH), jnp.bfloat16),
        pltpu.SemaphoreType.DMA,
        pltpu.SemaphoreType.DMA,
    ],
)(_sc_agg_body)


def _tc_comb_body(self_ref, dis_ref, q_ref, o_ref):
    acc = jnp.concatenate([q_ref[:N], q_ref[NPAD:NPAD + N]], axis=1)
    o_ref[...] = self_ref[...] + dis_ref[:N] * acc.astype(jnp.float32)


_tc_comb = pl.pallas_call(
    _tc_comb_body,
    out_shape=jax.ShapeDtypeStruct((N, D), jnp.float32),
)


def kernel(x, edge_index, self_weight, neighbor_weight, bias):
    npad_edges = ERP * D - E
    row_pad = jnp.concatenate(
        [edge_index[0], jnp.full((npad_edges,), NPAD - 1, jnp.int32)])
    col_pad = jnp.concatenate(
        [edge_index[1], jnp.zeros((npad_edges,), jnp.int32)])
    row2d = row_pad.reshape(ERP, D)
    col2d = col_pad.reshape(ERP, D)
    dis128 = _sc_deg(row2d)
    selfs, hps = _tc_main(x, self_weight, neighbor_weight,
                          bias.reshape(1, D), dis128)
    q = _sc_agg(hps, col2d, row2d)
    return _tc_comb(selfs, dis128, q)


# speedup vs baseline: 1.7852x; 1.0166x over previous
import functools

import jax
import jax.numpy as jnp
from jax import lax
from jax.experimental import pallas as pl
from jax.experimental.pallas import tpu as pltpu
from jax.experimental.pallas import tpu_sc as plsc

N = 10000
E = 320000
D = 128
DH = D // 2
NPAD = 10240
NC = 2
NS = 16
ER = E // D
ERP = 2560
CH = 128

_MESH = plsc.VectorSubcoreMesh(
    core_axis_name="c", subcore_axis_name="s", num_cores=NC, num_subcores=NS)

_DEG_ROWS = ERP // NS
_NODES_PER_SUB = NPAD // (NC * NS)


def _sc_deg_body(row2d, dis128, idx_v, ones_v, zro_v, dstage_v,
                 drows_v, deg_sh, sem):
    c = lax.axis_index("c")
    s = lax.axis_index("s")

    z16 = jnp.zeros((16,), jnp.float32)
    for k in range(640 // 16):
        zro_v[pl.ds(k * 16, 16)] = z16
    o16 = jnp.ones((16,), jnp.float32)
    for k in range(CH // 16):
        ones_v[pl.ds(k * 16, 16)] = o16

    pltpu.sync_copy(zro_v, deg_sh.at[pl.ds(s * 640, 640)])

    pltpu.sync_copy(row2d.at[pl.ds(s * _DEG_ROWS, _DEG_ROWS), :], idx_v)

    plsc.subcore_barrier()

    def _scatter(j, carry):
        pltpu.async_copy(ones_v, deg_sh.at[idx_v.at[j]], sem, add=True)

        @pl.when(j >= 4)
        def _():
            pltpu.make_async_copy(ones_v, deg_sh.at[idx_v.at[j]], sem).wait()
        return carry

    lax.fori_loop(0, _DEG_ROWS, _scatter, 0, unroll=False)
    for _ in range(4):
        pltpu.make_async_copy(ones_v, deg_sh.at[idx_v.at[0]], sem).wait()

    plsc.subcore_barrier()

    nbase = (c * NS + s) * _NODES_PER_SUB
    pltpu.sync_copy(deg_sh.at[pl.ds(nbase, _NODES_PER_SUB)], dstage_v)

    for v in range(_NODES_PER_SUB // 16):
        d = dstage_v[pl.ds(v * 16, 16)]
        i = lax.bitcast_convert_type(d, jnp.int32)
        i = jnp.int32(0x5F3759DF) - lax.shift_right_logical(i, 1)
        y = lax.bitcast_convert_type(i, jnp.float32)
        half = d * jnp.float32(0.5)
        for _ in range(3):
            y = y * (jnp.float32(1.5) - half * y * y)
        dstage_v[pl.ds(v * 16, 16)] = y

    def _splat(g, carry):
        v = dstage_v[pl.ds(g * 16, 16)]
        for r in range(16):
            v16 = lax.broadcast(v[r], (16,))
            for cc in range(D // 16):
                drows_v[r, pl.ds(cc * 16, 16)] = v16
        pltpu.sync_copy(drows_v, dis128.at[pl.ds(nbase + g * 16, 16), :])
        return carry

    lax.fori_loop(0, _NODES_PER_SUB // 16, _splat, 0, unroll=False)


_sc_deg = functools.partial(
    pl.kernel,
    out_type=jax.ShapeDtypeStruct((NPAD, D), jnp.float32),
    mesh=_MESH,
    scratch_types=[
        pltpu.VMEM((_DEG_ROWS, CH), jnp.int32),
        pltpu.VMEM((CH,), jnp.float32),
        pltpu.VMEM((640,), jnp.float32),
        pltpu.VMEM((_NODES_PER_SUB,), jnp.float32),
        pltpu.VMEM((16, D), jnp.float32),
        pltpu.VMEM_SHARED((NPAD,), jnp.float32),
        pltpu.SemaphoreType.DMA,
    ],
)(_sc_deg_body)


def _tc_main_body(x_ref, ws_ref, wn_ref, b_ref, dis_ref, self_ref, hps_ref):
    x = x_ref[...]
    self_ref[...] = (
        jnp.dot(x, ws_ref[...], preferred_element_type=jnp.float32)
        + b_ref[...])
    hp = dis_ref[:N] * jnp.dot(
        x, wn_ref[...], preferred_element_type=jnp.float32)
    hps_ref[0] = hp[:, :DH].astype(jnp.bfloat16)
    hps_ref[1] = hp[:, DH:].astype(jnp.bfloat16)


_tc_main = pl.pallas_call(
    _tc_main_body,
    out_shape=(
        jax.ShapeDtypeStruct((N, D), jnp.float32),
        jax.ShapeDtypeStruct((NC, N, DH), jnp.bfloat16),
    ),
)


_AGG_ROWS = ERP // NS
_AHALF = _AGG_ROWS // 2
_NBUF = 16
_G = 8
_L = _NBUF - _G


def _sc_agg_body(hps, col2d, row2d, q, cidx_v, ridx_v, rows_v, zblk_v,
                 acc_sh, gsem, ssem):
    c = lax.axis_index("c")
    s = lax.axis_index("s")

    b0 = s * _AGG_ROWS
    pltpu.async_copy(col2d.at[pl.ds(b0, _AGG_ROWS), :], cidx_v, gsem)
    pltpu.async_copy(row2d.at[pl.ds(b0, _AGG_ROWS), :], ridx_v, gsem)

    z32 = jnp.zeros((32,), jnp.bfloat16)
    for r in range(128):
        for cc in range(DH // 32):
            zblk_v[r, pl.ds(cc * 32, 32)] = z32

    for k in range(5):
        pltpu.async_copy(
            zblk_v, acc_sh.at[pl.ds(s * 640 + k * 128, 128), :], ssem)
    for k in range(5):
        pltpu.make_async_copy(
            zblk_v, acc_sh.at[pl.ds(s * 640, 128), :], ssem).wait()
    pltpu.make_async_copy(
        col2d.at[pl.ds(b0, _AGG_ROWS), :], cidx_v, gsem).wait()
    pltpu.make_async_copy(
        row2d.at[pl.ds(b0, _AGG_ROWS), :], ridx_v, gsem).wait()
    plsc.subcore_barrier()

    hpc = hps.at[c]

    for j0 in range(_G):
        pltpu.async_copy(hpc.at[cidx_v.at[j0]], rows_v.at[j0], gsem)

    def _agg8(jj, carry):
        for b in range(_NBUF):
            j = jj * _NBUF + b
            pltpu.make_async_copy(
                hpc.at[cidx_v.at[j]], rows_v.at[b], gsem).wait()
            pltpu.async_copy(
                rows_v.at[b], acc_sh.at[ridx_v.at[j]], ssem, add=True)

            @pl.when(j >= _L)
            def _():
                pltpu.make_async_copy(
                    rows_v.at[b], acc_sh.at[ridx_v.at[j]], ssem).wait()

            @pl.when(j + _G < _AGG_ROWS)
            def _():
                pltpu.async_copy(
                    hpc.at[cidx_v.at[j + _G]],
                    rows_v.at[(b + _G) % _NBUF], gsem)
        return carry

    lax.fori_loop(0, _AGG_ROWS // _NBUF, _agg8, 0, unroll=False)

    for _ in range(_L):
        pltpu.make_async_copy(
            rows_v.at[0], acc_sh.at[ridx_v.at[0]], ssem).wait()

    plsc.subcore_barrier()

    pltpu.sync_copy(acc_sh.at[pl.ds(s * 640, 640), :],
                    q.at[pl.ds(c * NPAD + s * 640, 640), :])


_sc_agg = functools.partial(
    pl.kernel,
    out_type=jax.ShapeDtypeStruct((NC * NPAD, DH), jnp.bfloat16),
    mesh=_MESH,
    compiler_params=pltpu.CompilerParams(use_tc_tiling_on_sc=False),
    scratch_types=[
        pltpu.VMEM((_AGG_ROWS, CH), jnp.int32),
        pltpu.VMEM((_AGG_ROWS, CH), jnp.int32),
        pltpu.VMEM((_NBUF, CH, DH), jnp.bfloat16),
        pltpu.VMEM((128, DH), jnp.bfloat16),
        pltpu.VMEM_SHARED((NPAD, DH), jnp.bfloat16),
        pltpu.SemaphoreType.DMA,
        pltpu.SemaphoreType.DMA,
    ],
)(_sc_agg_body)


def _tc_comb_body(self_ref, dis_ref, q_ref, o_ref):
    acc = jnp.concatenate([q_ref[:N], q_ref[NPAD:NPAD + N]], axis=1)
    o_ref[...] = self_ref[...] + dis_ref[:N] * acc.astype(jnp.float32)


_tc_comb = pl.pallas_call(
    _tc_comb_body,
    out_shape=jax.ShapeDtypeStruct((N, D), jnp.float32),
)


def kernel(x, edge_index, self_weight, neighbor_weight, bias):
    npad_edges = ERP * D - E
    row_pad = jnp.concatenate(
        [edge_index[0], jnp.full((npad_edges,), NPAD - 1, jnp.int32)])
    col_pad = jnp.concatenate(
        [edge_index[1], jnp.zeros((npad_edges,), jnp.int32)])
    row2d = row_pad.reshape(ERP, D)
    col2d = col_pad.reshape(ERP, D)
    dis128 = _sc_deg(row2d)
    selfs, hps = _tc_main(x, self_weight, neighbor_weight,
                          bias.reshape(1, D), dis128)
    q = _sc_agg(hps, col2d, row2d)
    return _tc_comb(selfs, dis128, q)
